# Initial kernel scaffold; baseline (speedup 1.0000x reference)
import jax
import jax.numpy as jnp
from jax.experimental import pallas as pl


def kernel(edge_indices, edges, edge_feat, node_embed, W1, b1, W2, b2, Wc, bc):
    raise NotImplementedError("write your pallas kernel here")


# speedup vs baseline: 9.8773x; 9.8773x over previous
import functools

import jax
import jax.numpy as jnp
from jax import lax
from jax.experimental import pallas as pl
from jax.experimental.pallas import tpu as pltpu
from jax.experimental.pallas import tpu_sc as plsc

NC, NS, L = 2, 16, 16
NW = NC * NS
NN = 10000
NB = 10240
NE = 320000
EPT = NE // NW
GRP = EPT // L
ROWS = 79
PAD = ROWS * 128 - EPT
BATCH = 8192
BPT = BATCH // NW
NPT = NB // NW

_mesh = plsc.VectorSubcoreMesh(core_axis_name="c", subcore_axis_name="s")
_cparams = pltpu.CompilerParams(needs_layout_passes=False,
                                use_tc_tiling_on_sc=False)
f32 = jnp.float32
i32 = jnp.int32


def _wid():
    return lax.axis_index("s") * NC + lax.axis_index("c")


def _zero_shared(zbuf, shared):
    sid = lax.axis_index("s")
    zeros = jnp.zeros((L,), f32)

    def zb(i, _):
        zbuf[pl.ds(i * L, L)] = zeros
        return 0

    lax.fori_loop(0, 640 // L, zb, 0)
    pltpu.sync_copy(zbuf, shared.at[pl.ds(sid * 640, 640)])


def _fill_pads(idx2d, val2d):
    iota = lax.iota(i32, L)
    zeros = jnp.zeros((L,), f32)
    for g in range(PAD // L):
        base = (GRP % 8) * L + g * L
        idx2d[ROWS - 1, pl.ds(base, L)] = NN + g * L + iota
        val2d[ROWS - 1, pl.ds(base, L)] = zeros


@functools.partial(
    pl.kernel,
    out_type=jax.ShapeDtypeStruct((NC * NB,), f32),
    mesh=_mesh,
    compiler_params=_cparams,
    scratch_types=[
        pltpu.VMEM((2 * EPT,), i32),
        pltpu.VMEM((ROWS, 128), i32),
        pltpu.VMEM((ROWS, 128), f32),
        pltpu.VMEM((640,), f32),
        pltpu.VMEM_SHARED((NB,), f32),
    ],
)
def _k1_deg(edges_hbm, degp_hbm, ebuf, idx2d, val2d, zbuf, shared):
    cid = lax.axis_index("c")
    sid = lax.axis_index("s")
    wid = _wid()
    _zero_shared(zbuf, shared)
    pltpu.sync_copy(edges_hbm.at[pl.ds(wid * 2 * EPT, 2 * EPT)], ebuf)
    _fill_pads(idx2d, val2d)
    plsc.subcore_barrier()

    iota = lax.iota(i32, L)
    ones = jnp.ones((L,), f32)

    def body(j, _):
        fj = j * (2 * L) + 2 * iota
        dstv = plsc.load_gather(ebuf, [fj + 1])
        r = j // 8
        k = (j % 8) * L
        idx2d[r, pl.ds(k, L)] = dstv
        val2d[r, pl.ds(k, L)] = ones
        return 0

    lax.fori_loop(0, GRP, body, 0)

    def scat(r, _):
        pltpu.sync_copy(val2d.at[r], shared.at[idx2d.at[r]], add=True)
        return 0

    lax.fori_loop(0, ROWS, scat, 0)
    plsc.subcore_barrier()
    pltpu.sync_copy(shared.at[pl.ds(sid * 640, 640)],
                    degp_hbm.at[pl.ds(cid * NB + sid * 640, 640)])


@functools.partial(
    pl.kernel,
    out_type=(jax.ShapeDtypeStruct((NC * NB,), f32),
              jax.ShapeDtypeStruct((NB,), f32)),
    mesh=_mesh,
    compiler_params=_cparams,
    scratch_types=[
        pltpu.VMEM((2 * EPT,), i32),
        pltpu.VMEM((NB,), f32),
        pltpu.VMEM((NB,), f32),
        pltpu.VMEM((ROWS, 128), i32),
        pltpu.VMEM((ROWS, 128), f32),
        pltpu.VMEM((640,), f32),
        pltpu.VMEM_SHARED((NB,), f32),
    ],
)
def _k2_t(edges_hbm, degp_hbm, tp_hbm, deg_hbm,
          ebuf, dl, db, idx2d, val2d, zbuf, shared):
    cid = lax.axis_index("c")
    sid = lax.axis_index("s")
    wid = _wid()
    _zero_shared(zbuf, shared)
    pltpu.sync_copy(degp_hbm.at[pl.ds(0, NB)], dl)
    pltpu.sync_copy(degp_hbm.at[pl.ds(NB, NB)], db)
    pltpu.sync_copy(edges_hbm.at[pl.ds(wid * 2 * EPT, 2 * EPT)], ebuf)
    _fill_pads(idx2d, val2d)

    def merge(i, _):
        s = pl.ds(i * L, L)
        dl[s] = dl[s] + db[s]
        return 0

    lax.fori_loop(0, NB // L, merge, 0)
    plsc.subcore_barrier()

    iota = lax.iota(i32, L)
    onesf = jnp.ones((L,), f32)
    zerosf = jnp.zeros((L,), f32)

    def body(j, _):
        fj = j * (2 * L) + 2 * iota
        srcv = plsc.load_gather(ebuf, [fj])
        dstv = plsc.load_gather(ebuf, [fj + 1])
        dsv = plsc.load_gather(dl, [srcv])
        val = jnp.where(dsv > 0.0, onesf, zerosf)
        r = j // 8
        k = (j % 8) * L
        idx2d[r, pl.ds(k, L)] = dstv
        val2d[r, pl.ds(k, L)] = val
        return 0

    lax.fori_loop(0, GRP, body, 0)

    def scat(r, _):
        pltpu.sync_copy(val2d.at[r], shared.at[idx2d.at[r]], add=True)
        return 0

    lax.fori_loop(0, ROWS, scat, 0)
    plsc.subcore_barrier()
    pltpu.sync_copy(shared.at[pl.ds(sid * 640, 640)],
                    tp_hbm.at[pl.ds(cid * NB + sid * 640, 640)])

    @pl.when(cid == 0)
    def _():
        pltpu.sync_copy(dl.at[pl.ds(sid * 640, 640)],
                        deg_hbm.at[pl.ds(sid * 640, 640)])


@functools.partial(
    pl.kernel,
    out_type=jax.ShapeDtypeStruct((4 * NB,), f32),
    mesh=_mesh,
    compiler_params=_cparams,
    scratch_types=[
        pltpu.VMEM((NPT,), f32),
        pltpu.VMEM((NPT,), f32),
        pltpu.VMEM((NPT,), f32),
        pltpu.VMEM((28, L), f32),
        pltpu.VMEM((NPT,), f32),
        pltpu.VMEM((NPT,), f32),
        pltpu.VMEM((NPT,), f32),
        pltpu.VMEM((NPT,), f32),
    ],
)
def _k3_tab(deg_hbm, tp_hbm, par_hbm, tab_hbm, dsl, t0, t1, par,
            ob0, ob1, ob2, ob3):
    wid = _wid()
    base = wid * NPT
    pltpu.sync_copy(deg_hbm.at[pl.ds(base, NPT)], dsl)
    pltpu.sync_copy(tp_hbm.at[pl.ds(base, NPT)], t0)
    pltpu.sync_copy(tp_hbm.at[pl.ds(NB + base, NPT)], t1)
    pltpu.sync_copy(par_hbm, par)

    ua = [par[j] for j in range(4)]
    ub = [par[4 + j] for j in range(4)]
    b2v = [par[8 + j] for j in range(4)]
    wp0 = [par[12 + j] for j in range(4)]
    wp1 = [par[16 + j] for j in range(4)]
    wq0 = [par[20 + j] for j in range(4)]
    wq1 = [par[24 + j] for j in range(4)]
    zeros = jnp.zeros((L,), f32)
    ones = jnp.ones((L,), f32)

    def body(g, _):
        s = pl.ds(g * L, L)
        dv = dsl[s]
        tv = t0[s] + t1[s]
        inv = ones / jnp.maximum(dv, 1.0)
        is0 = jnp.where(dv > 0.0, zeros, ones)
        av = is0 + (dv - tv) * inv
        bv = (ones - is0) + tv * inv
        h = [jnp.maximum(av * ua[j] + bv * ub[j] + b2v[j], 0.0)
             for j in range(4)]
        ob0[s] = h[0] * wp0[0] + h[1] * wp0[1] + h[2] * wp0[2] + h[3] * wp0[3]
        ob1[s] = h[0] * wp1[0] + h[1] * wp1[1] + h[2] * wp1[2] + h[3] * wp1[3]
        ob2[s] = h[0] * wq0[0] + h[1] * wq0[1] + h[2] * wq0[2] + h[3] * wq0[3]
        ob3[s] = h[0] * wq1[0] + h[1] * wq1[1] + h[2] * wq1[2] + h[3] * wq1[3]
        return 0

    lax.fori_loop(0, NPT // L, body, 0)
    for r, ob in enumerate((ob0, ob1, ob2, ob3)):
        pltpu.sync_copy(ob, tab_hbm.at[pl.ds(r * NB + base, NPT)])


@functools.partial(
    pl.kernel,
    out_type=jax.ShapeDtypeStruct((BATCH * 2,), f32),
    mesh=_mesh,
    compiler_params=_cparams,
    scratch_types=[
        pltpu.VMEM((2, 128), i32),
        pltpu.VMEM((2, 128), i32),
        pltpu.VMEM((2, 128), i32),
        pltpu.VMEM((2, 128), i32),
        pltpu.VMEM((2, 128), i32),
        pltpu.VMEM((2, 128, 16), f32),
        pltpu.VMEM((4 * NB,), f32),
        pltpu.VMEM((34, L), f32),
        pltpu.VMEM((2 * BPT,), f32),
        pltpu.SemaphoreType.DMA,
    ],
)
def _k4_batch(eidx_hbm, edges_hbm, ef_hbm, tab_hbm, par_hbm, out_hbm,
              ei2d, spos, dpos, srcid, dstid, efrows, tabv, par, ob, sem):
    wid = _wid()
    base = wid * BPT
    pltpu.sync_copy(tab_hbm, tabv)
    pltpu.sync_copy(par_hbm, par)
    for r in range(2):
        pltpu.sync_copy(eidx_hbm.at[pl.ds(base + r * 128, 128)], ei2d.at[r])

    iota = lax.iota(i32, L)
    for r in range(2):
        for g in range(8):
            s = pl.ds(g * L, L)
            ev = ei2d[r, s]
            spos[r, s] = 2 * ev
            dpos[r, s] = 2 * ev + 1
    cps = []
    for r in range(2):
        cps.append(pltpu.async_copy(
            edges_hbm.at[spos.at[r]], srcid.at[r], sem))
        cps.append(pltpu.async_copy(
            edges_hbm.at[dpos.at[r]], dstid.at[r], sem))
        cps.append(pltpu.async_copy(
            ef_hbm.at[ei2d.at[r]], efrows.at[r], sem))
    for cp in cps:
        cp.wait()

    we0 = [par[k] for k in range(16)]
    we1 = [par[16 + k] for k in range(16)]
    bc0 = par[32]
    bc1 = par[33]
    zer = jnp.zeros((L,), i32)
    one = jnp.ones((L,), i32)

    def body(g, _):
        r = g // 8
        rv = jnp.full((L,), r, i32)
        iv = (g % 8) * L + iota
        srcn = plsc.load_gather(srcid, [rv, iv])
        dstn = plsc.load_gather(dstid, [rv, iv])
        p0 = plsc.load_gather(tabv, [srcn])
        p1 = plsc.load_gather(tabv, [NB + srcn])
        q0 = plsc.load_gather(tabv, [2 * NB + dstn])
        q1 = plsc.load_gather(tabv, [3 * NB + dstn])
        a0 = bc0
        a1 = bc1
        for k in range(16):
            efk = plsc.load_gather(efrows, [rv, iv, jnp.full((L,), k, i32)])
            a0 = a0 + efk * we0[k]
            a1 = a1 + efk * we1[k]
        pos = 2 * ((g % 8) * L + 128 * r + iota)
        plsc.store_scatter(ob, [pos], p0 + q0 + a0)
        plsc.store_scatter(ob, [pos + 1], p1 + q1 + a1)
        return 0

    lax.fori_loop(0, 2 * 128 // L, body, 0)
    pltpu.sync_copy(ob, out_hbm.at[pl.ds(wid * 2 * BPT, 2 * BPT)])


def kernel(edge_indices, edges, edge_feat, node_embed, W1, b1, W2, b2, Wc, bc):
    edges_flat = edges.reshape(-1)

    s1 = jnp.sum(W1, axis=0)
    va = jnp.maximum(s1 + b1, 0.0)
    vb = jnp.maximum(2.0 * s1 + b1, 0.0)
    ua = va @ W2
    ub = vb @ W2
    par1 = jnp.concatenate(
        [ua, ub, b2, Wc[0:4, 0], Wc[0:4, 1], Wc[4:8, 0], Wc[4:8, 1]])
    par1 = jnp.tile(par1[:, None], (1, L))
    par2 = jnp.concatenate([Wc[8:24, 0], Wc[8:24, 1], bc])
    par2 = jnp.tile(par2[:, None], (1, L))

    degp = _k1_deg(edges_flat)
    tp, deg = _k2_t(edges_flat, degp)
    tab = _k3_tab(deg, tp, par1)
    out = _k4_batch(edge_indices, edges_flat, edge_feat, tab, par2)
    return out.reshape(BATCH, 2)


# speedup vs baseline: 10.3987x; 1.0528x over previous
import functools

import jax
import jax.numpy as jnp
from jax import lax
from jax.experimental import pallas as pl
from jax.experimental.pallas import tpu as pltpu
from jax.experimental.pallas import tpu_sc as plsc

NC, NS, L = 2, 16, 16
NW = NC * NS
NN = 10000
NB = 10240
NE = 320000
EPC = NE // NS
GRPC = EPC // L
ROWS = 157
PADS = ROWS * 128 - EPC
BATCH = 8192
BPT = BATCH // NW
SLC = NB // NS

_mesh = plsc.VectorSubcoreMesh(core_axis_name="c", subcore_axis_name="s")
_cparams = pltpu.CompilerParams(needs_layout_passes=False,
                                use_tc_tiling_on_sc=False)
f32 = jnp.float32
i32 = jnp.int32


@functools.partial(
    pl.kernel,
    out_type=jax.ShapeDtypeStruct((BATCH * 2,), f32),
    mesh=_mesh,
    compiler_params=_cparams,
    scratch_types=[
        pltpu.VMEM((2 * EPC,), i32),
        pltpu.VMEM((ROWS, 128), i32),
        pltpu.VMEM((ROWS, 128), f32),
        pltpu.VMEM((NB,), f32),
        pltpu.VMEM((640,), f32),
        pltpu.VMEM((640,), f32), pltpu.VMEM((640,), f32),
        pltpu.VMEM((640,), f32), pltpu.VMEM((640,), f32),
        pltpu.VMEM((2, 128), i32),
        pltpu.VMEM((2, 128), i32),
        pltpu.VMEM((2, 128), i32),
        pltpu.VMEM((2, 128), i32),
        pltpu.VMEM((2, 128), i32),
        pltpu.VMEM((4, 2, 128), i32),
        pltpu.VMEM((4, 2, 128), f32),
        pltpu.VMEM((16, 2, 128), i32),
        pltpu.VMEM((16, 2, 128), f32),
        pltpu.VMEM((62, L), f32),
        pltpu.VMEM((2 * BPT,), f32),
        pltpu.VMEM_SHARED((NB,), f32),
        pltpu.VMEM_SHARED((NB,), f32),
        pltpu.VMEM_SHARED((4 * NB,), f32),
        pltpu.SemaphoreType.DMA,
        pltpu.SemaphoreType.DMA,
        pltpu.SemaphoreType.DMA,
        pltpu.SemaphoreType.DMA,
    ],
)
def _k_all(eidx_hbm, edges_hbm, ef_hbm, par_hbm, out_hbm,
           ebuf, hidx, hval, dl, zbuf, tb0, tb1, tb2, tb3,
           ei2d, spos, dpos, srcid, dstid, pqidx, pqval, efpos, efval,
           par, ob, shd, sht, shtab, semh, seme, semf, semq):
    sid = lax.axis_index("s")
    wid = sid * NC + lax.axis_index("c")
    iota = lax.iota(i32, L)
    onesf = jnp.ones((L,), f32)
    zerosf = jnp.zeros((L,), f32)

    def zb(i, _):
        zbuf[pl.ds(i * L, L)] = zerosf
        return 0

    lax.fori_loop(0, 640 // L, zb, 0)
    pltpu.sync_copy(zbuf, shd.at[pl.ds(sid * 640, 640)])
    pltpu.sync_copy(zbuf, sht.at[pl.ds(sid * 640, 640)])
    pltpu.sync_copy(edges_hbm.at[pl.ds(sid * 2 * EPC, 2 * EPC)], ebuf)
    pltpu.sync_copy(par_hbm, par)

    for g in range(PADS // L):
        base = (GRPC % 8) * L + g * L
        hidx[ROWS - 1, pl.ds(base, L)] = NN + g * L + iota
        hval[ROWS - 1, pl.ds(base, L)] = zerosf
    plsc.subcore_barrier()

    def dein1(j, _):
        fj = j * (2 * L) + 2 * iota
        dstv = plsc.load_gather(ebuf, [fj + 1])
        r = j // 8
        k = (j % 8) * L
        hidx[r, pl.ds(k, L)] = dstv
        hval[r, pl.ds(k, L)] = onesf
        return 0

    lax.fori_loop(0, GRPC, dein1, 0)

    def fire(r, _):
        pltpu.async_copy(hval.at[r], shd.at[hidx.at[r]], semh, add=True)
        return 0

    def drain(r, _):
        pltpu.make_async_copy(hval.at[r], shd.at[hidx.at[r]], semh).wait()
        return 0

    lax.fori_loop(0, ROWS, fire, 0)
    lax.fori_loop(0, ROWS, drain, 0)
    plsc.subcore_barrier()

    pltpu.sync_copy(shd, dl)

    def dein2(j, _):
        fj = j * (2 * L) + 2 * iota
        srcv = plsc.load_gather(ebuf, [fj])
        dstv = plsc.load_gather(ebuf, [fj + 1])
        dsv = plsc.load_gather(dl, [srcv])
        r = j // 8
        k = (j % 8) * L
        hidx[r, pl.ds(k, L)] = dstv
        hval[r, pl.ds(k, L)] = jnp.where(dsv > 0.0, onesf, zerosf)
        return 0

    lax.fori_loop(0, GRPC, dein2, 0)

    def fire_t(r, _):
        pltpu.async_copy(hval.at[r], sht.at[hidx.at[r]], semh, add=True)
        return 0

    def drain_t(r, _):
        pltpu.make_async_copy(hval.at[r], sht.at[hidx.at[r]], semh).wait()
        return 0

    lax.fori_loop(0, ROWS, fire_t, 0)
    lax.fori_loop(0, ROWS, drain_t, 0)
    plsc.subcore_barrier()

    pltpu.sync_copy(sht.at[pl.ds(sid * 640, 640)], zbuf)
    ua = [par[j] for j in range(4)]
    ub = [par[4 + j] for j in range(4)]
    b2v = [par[8 + j] for j in range(4)]
    wp0 = [par[12 + j] for j in range(4)]
    wp1 = [par[16 + j] for j in range(4)]
    wq0 = [par[20 + j] for j in range(4)]
    wq1 = [par[24 + j] for j in range(4)]

    def tab(g, _):
        s = pl.ds(g * L, L)
        dv = dl[pl.ds(sid * 640 + g * L, L)]
        tv = zbuf[s]
        inv = onesf / jnp.maximum(dv, 1.0)
        is0 = jnp.where(dv > 0.0, zerosf, onesf)
        av = is0 + (dv - tv) * inv
        bv = (onesf - is0) + tv * inv
        h = [jnp.maximum(av * ua[j] + bv * ub[j] + b2v[j], 0.0)
             for j in range(4)]
        tb0[s] = h[0] * wp0[0] + h[1] * wp0[1] + h[2] * wp0[2] + h[3] * wp0[3]
        tb1[s] = h[0] * wp1[0] + h[1] * wp1[1] + h[2] * wp1[2] + h[3] * wp1[3]
        tb2[s] = h[0] * wq0[0] + h[1] * wq0[1] + h[2] * wq0[2] + h[3] * wq0[3]
        tb3[s] = h[0] * wq1[0] + h[1] * wq1[1] + h[2] * wq1[2] + h[3] * wq1[3]
        return 0

    lax.fori_loop(0, SLC // L, tab, 0)
    for r, tb in enumerate((tb0, tb1, tb2, tb3)):
        pltpu.sync_copy(tb, shtab.at[pl.ds(r * NB + sid * 640, 640)])
    plsc.subcore_barrier()

    bbase = wid * BPT
    for r in range(2):
        pltpu.sync_copy(eidx_hbm.at[pl.ds(bbase + r * 128, 128)], ei2d.at[r])

    def fill1(g, _):
        r = g // 8
        off = (g % 8) * L
        s = pl.ds(off, L)
        ev = ei2d[r, s]
        spos[r, s] = 2 * ev
        dpos[r, s] = 2 * ev + 1
        ev16 = 16 * ev
        for k in range(16):
            efpos[k, r, s] = ev16 + k
        return 0

    lax.fori_loop(0, 16, fill1, 0)
    ecps = []
    for r in range(2):
        ecps.append(pltpu.async_copy(edges_hbm.at[spos.at[r]],
                                     srcid.at[r], seme))
        ecps.append(pltpu.async_copy(edges_hbm.at[dpos.at[r]],
                                     dstid.at[r], seme))
    fcps = []
    for k in range(16):
        for r in range(2):
            fcps.append(pltpu.async_copy(ef_hbm.at[efpos.at[k, r]],
                                         efval.at[k, r], semf))
    for cp in ecps:
        cp.wait()

    def fill2(g, _):
        r = g // 8
        off = (g % 8) * L
        s = pl.ds(off, L)
        sv = srcid[r, s]
        dv = dstid[r, s]
        pqidx[0, r, s] = sv
        pqidx[1, r, s] = NB + sv
        pqidx[2, r, s] = 2 * NB + dv
        pqidx[3, r, s] = 3 * NB + dv
        return 0

    lax.fori_loop(0, 16, fill2, 0)
    qcps = []
    for t in range(4):
        for r in range(2):
            qcps.append(pltpu.async_copy(shtab.at[pqidx.at[t, r]],
                                         pqval.at[t, r], semq))
    for cp in fcps:
        cp.wait()
    for cp in qcps:
        cp.wait()

    we0 = [par[28 + k] for k in range(16)]
    we1 = [par[44 + k] for k in range(16)]
    bc0 = par[60]
    bc1 = par[61]

    def head(g, _):
        r = g // 8
        off = (g % 8) * L
        s = pl.ds(off, L)
        a0 = pqval[0, r, s] + pqval[2, r, s] + bc0
        a1 = pqval[1, r, s] + pqval[3, r, s] + bc1
        for k in range(16):
            efk = efval[k, r, s]
            a0 = a0 + efk * we0[k]
            a1 = a1 + efk * we1[k]
        pos = 2 * (r * 128 + off + iota)
        plsc.store_scatter(ob, [pos], a0)
        plsc.store_scatter(ob, [pos + 1], a1)
        return 0

    lax.fori_loop(0, 16, head, 0)
    pltpu.sync_copy(ob, out_hbm.at[pl.ds(wid * 2 * BPT, 2 * BPT)])


def kernel(edge_indices, edges, edge_feat, node_embed, W1, b1, W2, b2, Wc, bc):
    edges_flat = edges.reshape(-1)
    ef_flat = edge_feat.reshape(-1)

    s1 = jnp.sum(W1, axis=0)
    va = jnp.maximum(s1 + b1, 0.0)
    vb = jnp.maximum(2.0 * s1 + b1, 0.0)
    ua = va @ W2
    ub = vb @ W2
    par = jnp.concatenate(
        [ua, ub, b2, Wc[0:4, 0], Wc[0:4, 1], Wc[4:8, 0], Wc[4:8, 1],
         Wc[8:24, 0], Wc[8:24, 1], bc])
    par = jnp.tile(par[:, None], (1, L))

    out = _k_all(edge_indices, edges_flat, ef_flat, par)
    return out.reshape(BATCH, 2)


# speedup vs baseline: 38.2931x; 3.6825x over previous
import functools

import jax
import jax.numpy as jnp
from jax import lax
from jax.experimental import pallas as pl
from jax.experimental.pallas import tpu as pltpu
from jax.experimental.pallas import tpu_sc as plsc

NC, NS, L = 2, 16, 16
NW = NC * NS
NN = 10000
NB = 10240
NE = 320000
EPC = NE // NS
GRPC = EPC // L
ROWS = 157
PADS = ROWS * 128 - EPC
BATCH = 8192
BPT = BATCH // NW
SLC = NB // NS

_mesh = plsc.VectorSubcoreMesh(core_axis_name="c", subcore_axis_name="s")
_cparams = pltpu.CompilerParams(needs_layout_passes=False,
                                use_tc_tiling_on_sc=False)
f32 = jnp.float32
i32 = jnp.int32


@functools.partial(
    pl.kernel,
    out_type=jax.ShapeDtypeStruct((BATCH * 2,), f32),
    mesh=_mesh,
    compiler_params=_cparams,
    scratch_types=[
        pltpu.VMEM((EPC,), i32),
        pltpu.VMEM((EPC,), i32),
        pltpu.VMEM((ROWS, 128), i32),
        pltpu.VMEM((ROWS, 128), f32),
        pltpu.VMEM((NB,), f32),
        pltpu.VMEM((640,), f32),
        pltpu.VMEM((640,), f32), pltpu.VMEM((640,), f32),
        pltpu.VMEM((640,), f32), pltpu.VMEM((640,), f32),
        pltpu.VMEM((2, 128), i32),
        pltpu.VMEM((2, 128), i32),
        pltpu.VMEM((2, 128), i32),
        pltpu.VMEM((4, 2, 128), i32),
        pltpu.VMEM((4, 2, 128), f32),
        pltpu.VMEM((16, 2, 128), i32),
        pltpu.VMEM((16, 2, 128), f32),
        pltpu.VMEM((62, L), f32),
        pltpu.VMEM((2 * BPT,), f32),
        pltpu.VMEM_SHARED((NB,), f32),
        pltpu.VMEM_SHARED((NB,), f32),
        pltpu.VMEM_SHARED((4 * NB,), f32),
        pltpu.SemaphoreType.DMA,
        pltpu.SemaphoreType.DMA,
        pltpu.SemaphoreType.DMA,
        pltpu.SemaphoreType.DMA,
    ],
)
def _k_all(eidx_hbm, srccol_hbm, dstcol_hbm, ef_hbm, par_hbm, out_hbm,
           sbuf, dbuf, hidx, hval, dl, zbuf, tb0, tb1, tb2, tb3,
           ei2d, srcid, dstid, pqidx, pqval, efpos, efval,
           par, ob, shd, sht, shtab, semh, seme, semf, semq):
    sid = lax.axis_index("s")
    wid = sid * NC + lax.axis_index("c")
    iota = lax.iota(i32, L)
    onesf = jnp.ones((L,), f32)
    zerosf = jnp.zeros((L,), f32)

    def zb(i, _):
        zbuf[pl.ds(i * L, L)] = zerosf
        return 0

    lax.fori_loop(0, 640 // L, zb, 0)
    pltpu.sync_copy(zbuf, shd.at[pl.ds(sid * 640, 640)])
    pltpu.sync_copy(zbuf, sht.at[pl.ds(sid * 640, 640)])
    pltpu.sync_copy(srccol_hbm.at[pl.ds(sid * EPC, EPC)], sbuf)
    pltpu.sync_copy(dstcol_hbm.at[pl.ds(sid * EPC, EPC)], dbuf)
    pltpu.sync_copy(par_hbm, par)

    for g in range(PADS // L):
        base = (GRPC % 8) * L + g * L
        hidx[ROWS - 1, pl.ds(base, L)] = NN + g * L + iota
        hval[ROWS - 1, pl.ds(base, L)] = zerosf
    plsc.subcore_barrier()

    zeroi = jnp.zeros((L,), i32)
    onei = jnp.ones((L,), i32)

    def dein1(j, _):
        dstv = dbuf[pl.ds(j * L, L)]
        r = j // 8
        k = (j % 8) * L
        hidx[r, pl.ds(k, L)] = dstv
        hval[r, pl.ds(k, L)] = onesf
        return 0

    lax.fori_loop(0, GRPC, dein1, 0)

    def fire(r, _):
        pltpu.async_copy(hval.at[r], shd.at[hidx.at[r]], semh, add=True)
        return 0

    def drain(r, _):
        pltpu.make_async_copy(hval.at[r], shd.at[hidx.at[r]], semh).wait()
        return 0

    lax.fori_loop(0, ROWS, fire, 0)
    lax.fori_loop(0, ROWS, drain, 0)
    plsc.subcore_barrier()

    pltpu.sync_copy(shd, dl)

    def dein2(j, _):
        srcv = sbuf[pl.ds(j * L, L)]
        dstv = dbuf[pl.ds(j * L, L)]
        dsv = plsc.load_gather(dl, [srcv])
        r = j // 8
        k = (j % 8) * L
        hidx[r, pl.ds(k, L)] = dstv
        hval[r, pl.ds(k, L)] = jnp.where(dsv > 0.0, onesf, zerosf)
        return 0

    lax.fori_loop(0, GRPC, dein2, 0)

    def fire_t(r, _):
        pltpu.async_copy(hval.at[r], sht.at[hidx.at[r]], semh, add=True)
        return 0

    def drain_t(r, _):
        pltpu.make_async_copy(hval.at[r], sht.at[hidx.at[r]], semh).wait()
        return 0

    lax.fori_loop(0, ROWS, fire_t, 0)
    lax.fori_loop(0, ROWS, drain_t, 0)
    plsc.subcore_barrier()

    pltpu.sync_copy(sht.at[pl.ds(sid * 640, 640)], zbuf)
    ua = [par[j] for j in range(4)]
    ub = [par[4 + j] for j in range(4)]
    b2v = [par[8 + j] for j in range(4)]
    wp0 = [par[12 + j] for j in range(4)]
    wp1 = [par[16 + j] for j in range(4)]
    wq0 = [par[20 + j] for j in range(4)]
    wq1 = [par[24 + j] for j in range(4)]

    def tab(g, _):
        s = pl.ds(g * L, L)
        dv = dl[pl.ds(sid * 640 + g * L, L)]
        tv = zbuf[s]
        inv = onesf / jnp.maximum(dv, 1.0)
        is0 = jnp.where(dv > 0.0, zerosf, onesf)
        av = is0 + (dv - tv) * inv
        bv = (onesf - is0) + tv * inv
        h = [jnp.maximum(av * ua[j] + bv * ub[j] + b2v[j], 0.0)
             for j in range(4)]
        tb0[s] = h[0] * wp0[0] + h[1] * wp0[1] + h[2] * wp0[2] + h[3] * wp0[3]
        tb1[s] = h[0] * wp1[0] + h[1] * wp1[1] + h[2] * wp1[2] + h[3] * wp1[3]
        tb2[s] = h[0] * wq0[0] + h[1] * wq0[1] + h[2] * wq0[2] + h[3] * wq0[3]
        tb3[s] = h[0] * wq1[0] + h[1] * wq1[1] + h[2] * wq1[2] + h[3] * wq1[3]
        return 0

    lax.fori_loop(0, SLC // L, tab, 0)
    for r, tb in enumerate((tb0, tb1, tb2, tb3)):
        pltpu.sync_copy(tb, shtab.at[pl.ds(r * NB + sid * 640, 640)])
    plsc.subcore_barrier()

    bbase = wid * BPT
    for r in range(2):
        pltpu.sync_copy(eidx_hbm.at[pl.ds(bbase + r * 128, 128)], ei2d.at[r])

    def fillef(g, _):
        r = g // 8
        off = (g % 8) * L
        s = pl.ds(off, L)
        ev = ei2d[r, s]
        for k in range(16):
            efpos[k, r, s] = k * NE + ev
        return 0

    lax.fori_loop(0, 16, fillef, 0)

    ecps = []
    for r in range(2):
        ecps.append(pltpu.async_copy(srccol_hbm.at[ei2d.at[r]],
                                     srcid.at[r], seme))
        ecps.append(pltpu.async_copy(dstcol_hbm.at[ei2d.at[r]],
                                     dstid.at[r], seme))

    def fireef(i, _):
        pltpu.async_copy(ef_hbm.at[efpos.at[i // 2, i % 2]],
                         efval.at[i // 2, i % 2], semf)
        return 0

    lax.fori_loop(0, 32, fireef, 0)
    for cp in ecps:
        cp.wait()

    def fill2(g, _):
        r = g // 8
        off = (g % 8) * L
        s = pl.ds(off, L)
        sv = srcid[r, s]
        dv = dstid[r, s]
        pqidx[0, r, s] = sv
        pqidx[1, r, s] = NB + sv
        pqidx[2, r, s] = 2 * NB + dv
        pqidx[3, r, s] = 3 * NB + dv
        return 0

    lax.fori_loop(0, 16, fill2, 0)
    qcps = []
    for t in range(4):
        for r in range(2):
            qcps.append(pltpu.async_copy(shtab.at[pqidx.at[t, r]],
                                         pqval.at[t, r], semq))

    def drainef(i, _):
        pltpu.make_async_copy(ef_hbm.at[efpos.at[i // 2, i % 2]],
                              efval.at[i // 2, i % 2], semf).wait()
        return 0

    lax.fori_loop(0, 32, drainef, 0)
    for cp in qcps:
        cp.wait()

    we0 = [par[28 + k] for k in range(16)]
    we1 = [par[44 + k] for k in range(16)]
    bc0 = par[60]
    bc1 = par[61]

    def head(g, _):
        r = g // 8
        off = (g % 8) * L
        s = pl.ds(off, L)
        a0 = pqval[0, r, s] + pqval[2, r, s] + bc0
        a1 = pqval[1, r, s] + pqval[3, r, s] + bc1
        for k in range(16):
            efk = efval[k, r, s]
            a0 = a0 + efk * we0[k]
            a1 = a1 + efk * we1[k]
        pos = 2 * (r * 128 + off + iota)
        plsc.store_scatter(ob, [pos], a0)
        plsc.store_scatter(ob, [pos + 1], a1)
        return 0

    lax.fori_loop(0, 16, head, 0)
    pltpu.sync_copy(ob, out_hbm.at[pl.ds(wid * 2 * BPT, 2 * BPT)])


def kernel(edge_indices, edges, edge_feat, node_embed, W1, b1, W2, b2, Wc, bc):
    s1 = jnp.sum(W1, axis=0)
    va = jnp.maximum(s1 + b1, 0.0)
    vb = jnp.maximum(2.0 * s1 + b1, 0.0)
    ua = va @ W2
    ub = vb @ W2
    par = jnp.concatenate(
        [ua, ub, b2, Wc[0:4, 0], Wc[0:4, 1], Wc[4:8, 0], Wc[4:8, 1],
         Wc[8:24, 0], Wc[8:24, 1], bc])
    par = jnp.tile(par[:, None], (1, L))

    eft = edge_feat.T.reshape(-1)
    out = _k_all(edge_indices, edges[:, 0], edges[:, 1], eft, par)
    return out.reshape(BATCH, 2)
